# baseline (speedup 1.0000x reference)
import jax
import jax.numpy as jnp
import numpy as np
from jax.experimental import pallas as pl
from jax.experimental.pallas import tpu as pltpu
from jax.experimental.pallas import tpu_sc as plsc

_B = 2048
_W = 256
_G = 4


def _sc_gather(nodes2, idx, ep):
    feat = nodes2.shape[1]
    idx2 = idx.reshape(1, ep)
    mesh = plsc.VectorSubcoreMesh(core_axis_name="core", subcore_axis_name="subcore")

    @pl.kernel(out_type=jax.ShapeDtypeStruct((ep, feat), nodes2.dtype), mesh=mesh)
    def gather_kernel(x_hbm, i_hbm, o_hbm):
        def body(i_vmem, o_vmem):
            pltpu.sync_copy(x_hbm.at[i_vmem.at[0]], o_vmem)

        pltpu.emit_pipeline(
            body,
            grid=(ep // _W,),
            in_specs=[pl.BlockSpec((1, _W), index_map=lambda i: (0, i))],
            out_specs=[pl.BlockSpec((_W, feat), index_map=lambda i: (i, 0))],
            core_axis_name=("core", "subcore"),
            dimension_semantics=(pltpu.PARALLEL,),
        )(i_hbm, o_hbm)

    return gather_kernel(nodes2, idx2)


def _regroup(kbig, g, num_rel, out_f):
    wide = kbig.shape[0]
    s = num_rel // g
    return (kbig.reshape(wide, g, s * out_f)
            .transpose(1, 0, 2)
            .reshape(g * wide, s * out_f))


def _tc_messages(x128, par2, types2, kgrp, ep, in_f, rf, out_f):
    nb = ep // _B
    wide = 2 * in_f
    s = rf // out_f // _G
    yw = s * out_f

    def body(x_ref, p_ref, t_ref, k_ref, o_ref):
        xw = x_ref[...]
        pb = p_ref[...]
        tb = t_ref[...]
        col = jax.lax.broadcasted_iota(jnp.int32, (_B, wide), 1)
        xm = jnp.where((col // in_f) == pb, xw, 0.0).astype(jnp.bfloat16)
        xg = jnp.concatenate([xm] * _G, axis=1)
        gcol = jax.lax.broadcasted_iota(jnp.int32, (_B, wide * _G), 1)
        xg = jnp.where((gcol // wide) == (tb // s), xg, jnp.bfloat16(0))
        y = jnp.dot(xg, k_ref[...], preferred_element_type=jnp.float32)
        jcol = jax.lax.broadcasted_iota(jnp.int32, (_B, yw), 1)
        ym = jnp.where((jcol // out_f) == (tb % s), y, 0.0)
        acc = ym[:, 0:128]
        for c in range(1, yw // 128):
            acc = acc + ym[:, 128 * c:128 * (c + 1)]
        res = acc[:, 0:out_f]
        for j in range(1, 128 // out_f):
            res = res + acc[:, out_f * j:out_f * (j + 1)]
        o_ref[...] = res

    return pl.pallas_call(
        body,
        grid=(nb,),
        in_specs=[
            pl.BlockSpec((_B, wide), lambda i: (i, 0)),
            pl.BlockSpec((_B, 1), lambda i: (i, 0)),
            pl.BlockSpec((_B, 1), lambda i: (i, 0)),
            pl.BlockSpec((wide * _G, yw), lambda i: (0, 0)),
        ],
        out_specs=pl.BlockSpec((_B, out_f), lambda i: (i, 0)),
        out_shape=jax.ShapeDtypeStruct((ep, out_f), jnp.float32),
        compiler_params=pltpu.CompilerParams(
            dimension_semantics=("parallel",)),
    )(x128, par2, types2, kgrp)


def kernel(nodes, senders, edge_types, kernels):
    e = senders.shape[0]
    num_rel, in_f, out_f = kernels.shape
    rf = num_rel * out_f
    nodes2 = nodes.reshape(nodes.shape[0] // 2, 2 * in_f)

    lcm = int(np.lcm(_B, _W))
    ep = ((e + lcm - 1) // lcm) * lcm
    pad = ep - e
    sp = jnp.pad(senders, (0, pad))
    tp = jnp.pad(edge_types, (0, pad))

    x128 = _sc_gather(nodes2, sp >> 1, ep)

    kflat = jnp.transpose(kernels, (1, 0, 2)).reshape(in_f, rf)
    kbig = jnp.concatenate([kflat, kflat], axis=0).astype(jnp.bfloat16)
    kgrp = _regroup(kbig, _G, num_rel, out_f)

    out = _tc_messages(x128, (sp & 1).reshape(ep, 1),
                       tp.reshape(ep, 1), kgrp, ep, in_f, rf, out_f)
    return out[:e]


# speedup vs baseline: 1.2494x; 1.2494x over previous
import jax
import jax.numpy as jnp
import numpy as np
from jax.experimental import pallas as pl
from jax.experimental.pallas import tpu as pltpu
from jax.experimental.pallas import tpu_sc as plsc

_B = 2048
_W = 128
_G = 4
_SCV = 16


def _sc_gather(nodes2, senders, ep):
    feat = nodes2.shape[1]
    idx2 = senders.reshape(1, ep)
    mesh = plsc.VectorSubcoreMesh(core_axis_name="core", subcore_axis_name="subcore")

    @pl.kernel(out_type=jax.ShapeDtypeStruct((ep, feat), nodes2.dtype),
               mesh=mesh,
               scratch_types=[pltpu.VMEM((1, _W), jnp.int32)])
    def gather_kernel(x_hbm, i_hbm, o_hbm, half_idx):
        def body(i_vmem, o_vmem):
            @pl.loop(0, _W, step=_SCV)
            def _(c):
                sl = (slice(0, 1), pl.ds(c, _SCV))
                half_idx[sl] = i_vmem[sl] >> 1

            pltpu.sync_copy(x_hbm.at[half_idx.at[0]], o_vmem)

        pltpu.emit_pipeline(
            body,
            grid=(ep // _W,),
            in_specs=[pl.BlockSpec((1, _W), index_map=lambda i: (0, i))],
            out_specs=[pl.BlockSpec((_W, feat), index_map=lambda i: (i, 0))],
            core_axis_name=("core", "subcore"),
            dimension_semantics=(pltpu.PARALLEL,),
        )(i_hbm, o_hbm)

    return gather_kernel(nodes2, idx2)


def _regroup(kbig, g, num_rel, out_f):
    wide = kbig.shape[0]
    s = num_rel // g
    return (kbig.reshape(wide, g, s * out_f)
            .transpose(1, 0, 2)
            .reshape(g * wide, s * out_f))


def _tc_messages(x128, send3, type3, kgrp, e, ep, in_f, rf, out_f):
    nb = ep // _B
    wide = 2 * in_f
    s = rf // out_f // _G
    yw = s * out_f

    def body(x_ref, s_ref, t_ref, k_ref, o_ref):
        xw = x_ref[...]
        sv = s_ref[0]
        tv = t_ref[0]
        codev = (tv << 1) | (sv & 1)
        code = codev.reshape(_B, 1)
        slot = ((code >> 1) // s) * 2 + (code & 1)
        jrel = (code >> 1) % s
        xb = xw.astype(jnp.bfloat16)
        xg = jnp.concatenate([xb] * _G, axis=1)
        gcol = jax.lax.broadcasted_iota(jnp.int32, (_B, wide * _G), 1)
        xg = jnp.where((gcol // in_f) == slot, xg, jnp.bfloat16(0))
        y = jnp.dot(xg, k_ref[...], preferred_element_type=jnp.float32)
        jcol = jax.lax.broadcasted_iota(jnp.int32, (_B, yw), 1)
        ym = jnp.where((jcol // out_f) == jrel, y, 0.0)
        acc = ym[:, 0:128]
        for c in range(1, yw // 128):
            acc = acc + ym[:, 128 * c:128 * (c + 1)]
        res = acc[:, 0:out_f]
        for j in range(1, 128 // out_f):
            res = res + acc[:, out_f * j:out_f * (j + 1)]
        o_ref[...] = res

    return pl.pallas_call(
        body,
        grid=(nb,),
        in_specs=[
            pl.BlockSpec((_B, wide), lambda i: (i, 0)),
            pl.BlockSpec((1, 1, _B), lambda i: (i, 0, 0)),
            pl.BlockSpec((1, 1, _B), lambda i: (i, 0, 0)),
            pl.BlockSpec((_G * wide, yw), lambda i: (0, 0)),
        ],
        out_specs=pl.BlockSpec((_B, out_f), lambda i: (i, 0)),
        out_shape=jax.ShapeDtypeStruct((e, out_f), jnp.float32),
    )(x128, send3, type3, kgrp)


def kernel(nodes, senders, edge_types, kernels):
    e = senders.shape[0]
    num_rel, in_f, out_f = kernels.shape
    rf = num_rel * out_f
    nodes2 = nodes.reshape(nodes.shape[0] // 2, 2 * in_f)

    lcm = int(np.lcm(_B, _W))
    ep = ((e + lcm - 1) // lcm) * lcm
    pad = ep - e
    sp = jnp.pad(senders, (0, pad))
    tp = jnp.pad(edge_types, (0, pad))
    nb = ep // _B

    x128 = _sc_gather(nodes2, sp, ep)

    kflat = jnp.transpose(kernels, (1, 0, 2)).reshape(in_f, rf)
    kbig = jnp.concatenate([kflat, kflat], axis=0).astype(jnp.bfloat16)
    kgrp = _regroup(kbig, _G, num_rel, out_f)

    out = _tc_messages(x128, sp.reshape(nb, 1, _B), tp.reshape(nb, 1, _B),
                       kgrp, e, ep, in_f, rf, out_f)
    return out


# speedup vs baseline: 1.3029x; 1.0428x over previous
import jax
import jax.numpy as jnp
import numpy as np
from jax.experimental import pallas as pl
from jax.experimental.pallas import tpu as pltpu
from jax.experimental.pallas import tpu_sc as plsc

_B = 2048
_W = 128
_G = 4
_SCV = 16


def _sc_gather(nodes2, senders, ep):
    feat = nodes2.shape[1]
    idx2 = senders.reshape(1, ep)
    mesh = plsc.VectorSubcoreMesh(core_axis_name="core", subcore_axis_name="subcore")

    @pl.kernel(out_type=jax.ShapeDtypeStruct((ep, feat), nodes2.dtype),
               mesh=mesh,
               scratch_types=[pltpu.VMEM((1, _W), jnp.int32)])
    def gather_kernel(x_hbm, i_hbm, o_hbm, half_idx):
        def body(i_vmem, o_vmem):
            @pl.loop(0, _W, step=_SCV)
            def _(c):
                sl = (slice(0, 1), pl.ds(c, _SCV))
                half_idx[sl] = i_vmem[sl] >> 1

            pltpu.sync_copy(x_hbm.at[half_idx.at[0]], o_vmem)

        pltpu.emit_pipeline(
            body,
            grid=(ep // _W,),
            in_specs=[pl.BlockSpec((1, _W), index_map=lambda i: (0, i))],
            out_specs=[pl.BlockSpec((_W, feat), index_map=lambda i: (i, 0))],
            core_axis_name=("core", "subcore"),
            dimension_semantics=(pltpu.PARALLEL,),
        )(i_hbm, o_hbm)

    return gather_kernel(nodes2, idx2)


def _regroup(kbig, g, num_rel, out_f):
    wide = kbig.shape[0]
    s = num_rel // g
    return (kbig.reshape(wide, g, s * out_f)
            .transpose(1, 0, 2)
            .reshape(g * wide, s * out_f))


def _tc_messages(x128, send3, type3, kgrp, e, ep, in_f, rf, out_f):
    nb = ep // _B
    wide = 2 * in_f
    s = rf // out_f // _G
    yw = s * out_f

    def body(x_ref, s_ref, t_ref, k_ref, o_ref):
        xw = x_ref[...]
        sv = s_ref[0]
        tv = t_ref[0]
        codev = (tv << 1) | (sv & 1)
        code = codev.reshape(_B, 1)
        slot = ((code >> 1) // s) * 2 + (code & 1)
        jrel = (code >> 1) % s
        xb = xw.astype(jnp.bfloat16)
        xg = jnp.concatenate([xb] * _G, axis=1)
        gcol = jax.lax.broadcasted_iota(jnp.int32, (_B, wide * _G), 1)
        xg = jnp.where((gcol // in_f) == slot, xg, jnp.bfloat16(0))
        y = jnp.dot(xg, k_ref[...], preferred_element_type=jnp.float32)
        jcol = jax.lax.broadcasted_iota(jnp.int32, (_B, yw), 1)
        ym = jnp.where((jcol // out_f) == jrel, y, 0.0)
        acc = ym[:, 0:128]
        for c in range(1, yw // 128):
            acc = acc + ym[:, 128 * c:128 * (c + 1)]
        res = acc[:, 0:out_f]
        for j in range(1, 128 // out_f):
            res = res + acc[:, out_f * j:out_f * (j + 1)]
        o_ref[...] = jnp.swapaxes(res, 0, 1)

    return pl.pallas_call(
        body,
        grid=(nb,),
        in_specs=[
            pl.BlockSpec((_B, wide), lambda i: (i, 0)),
            pl.BlockSpec((1, 1, _B), lambda i: (i, 0, 0)),
            pl.BlockSpec((1, 1, _B), lambda i: (i, 0, 0)),
            pl.BlockSpec((_G * wide, yw), lambda i: (0, 0)),
        ],
        out_specs=pl.BlockSpec((out_f, _B), lambda i: (0, i)),
        out_shape=jax.ShapeDtypeStruct((out_f, e), jnp.float32),
    )(x128, send3, type3, kgrp)


def kernel(nodes, senders, edge_types, kernels):
    e = senders.shape[0]
    num_rel, in_f, out_f = kernels.shape
    rf = num_rel * out_f
    nodes2 = nodes.reshape(nodes.shape[0] // 2, 2 * in_f)

    lcm = int(np.lcm(_B, _W))
    ep = ((e + lcm - 1) // lcm) * lcm
    pad = ep - e
    sp = jnp.pad(senders, (0, pad))
    tp = jnp.pad(edge_types, (0, pad))
    nb = ep // _B

    x128 = _sc_gather(nodes2, sp, ep)

    kflat = jnp.transpose(kernels, (1, 0, 2)).reshape(in_f, rf)
    kbig = jnp.concatenate([kflat, kflat], axis=0).astype(jnp.bfloat16)
    kgrp = _regroup(kbig, _G, num_rel, out_f)

    out = _tc_messages(x128, sp.reshape(nb, 1, _B), tp.reshape(nb, 1, _B),
                       kgrp, e, ep, in_f, rf, out_f)
    return out.T


# speedup vs baseline: 1.3121x; 1.0071x over previous
import jax
import jax.numpy as jnp
import numpy as np
from jax.experimental import pallas as pl
from jax.experimental.pallas import tpu as pltpu
from jax.experimental.pallas import tpu_sc as plsc

_B = 2048
_W = 128
_G = 4
_SCV = 16


def _sc_gather(nodes2, idx, ep):
    feat = nodes2.shape[1]
    idx2 = idx.reshape(1, ep)
    mesh = plsc.VectorSubcoreMesh(core_axis_name="core", subcore_axis_name="subcore")

    @pl.kernel(out_type=jax.ShapeDtypeStruct((ep, feat), nodes2.dtype),
               mesh=mesh)
    def gather_kernel(x_hbm, i_hbm, o_hbm):
        def body(i_vmem, o_vmem):
            pltpu.sync_copy(x_hbm.at[i_vmem.at[0]], o_vmem)

        pltpu.emit_pipeline(
            body,
            grid=(ep // _W,),
            in_specs=[pl.BlockSpec((1, _W), index_map=lambda i: (0, i))],
            out_specs=[pl.BlockSpec((_W, feat), index_map=lambda i: (i, 0))],
            core_axis_name=("core", "subcore"),
            dimension_semantics=(pltpu.PARALLEL,),
        )(i_hbm, o_hbm)

    return gather_kernel(nodes2, idx2)


def _regroup(kbig, g, num_rel, out_f):
    wide = kbig.shape[0]
    s = num_rel // g
    return (kbig.reshape(wide, g, s * out_f)
            .transpose(1, 0, 2)
            .reshape(g * wide, s * out_f))


def _tc_messages(x128, send3, type3, kgrp, e, ep, in_f, rf, out_f):
    nb = ep // _B
    wide = 2 * in_f
    s = rf // out_f // _G
    yw = s * out_f

    def body(x_ref, s_ref, t_ref, k_ref, o_ref):
        xw = x_ref[...]
        sv = s_ref[0]
        tv = t_ref[0]
        codev = (tv << 1) | (sv & 1)
        code = codev.reshape(_B, 1)
        slot = ((code >> 1) // s) * 2 + (code & 1)
        jrel = (code >> 1) % s
        xb = xw.astype(jnp.bfloat16)
        xg = jnp.concatenate([xb] * _G, axis=1)
        gcol = jax.lax.broadcasted_iota(jnp.int32, (_B, wide * _G), 1)
        xg = jnp.where((gcol // in_f) == slot, xg, jnp.bfloat16(0))
        y = jnp.dot(xg, k_ref[...], preferred_element_type=jnp.float32)
        jcol = jax.lax.broadcasted_iota(jnp.int32, (_B, yw), 1)
        ym = jnp.where((jcol // out_f) == jrel, y, 0.0)
        acc = ym[:, 0:128]
        for c in range(1, yw // 128):
            acc = acc + ym[:, 128 * c:128 * (c + 1)]
        res = acc[:, 0:out_f]
        for j in range(1, 128 // out_f):
            res = res + acc[:, out_f * j:out_f * (j + 1)]
        o_ref[...] = jnp.swapaxes(res, 0, 1)

    return pl.pallas_call(
        body,
        grid=(nb,),
        in_specs=[
            pl.BlockSpec((_B, wide), lambda i: (i, 0)),
            pl.BlockSpec((1, 1, _B), lambda i: (i, 0, 0)),
            pl.BlockSpec((1, 1, _B), lambda i: (i, 0, 0)),
            pl.BlockSpec((_G * wide, yw), lambda i: (0, 0)),
        ],
        out_specs=pl.BlockSpec((out_f, _B), lambda i: (0, i)),
        out_shape=jax.ShapeDtypeStruct((out_f, e), jnp.float32),
    )(x128, send3, type3, kgrp)


def kernel(nodes, senders, edge_types, kernels):
    e = senders.shape[0]
    num_rel, in_f, out_f = kernels.shape
    rf = num_rel * out_f
    nodes2 = nodes.reshape(nodes.shape[0] // 2, 2 * in_f)

    lcm = int(np.lcm(_B, _W))
    ep = ((e + lcm - 1) // lcm) * lcm
    pad = ep - e
    sp = jnp.pad(senders, (0, pad))
    tp = jnp.pad(edge_types, (0, pad))
    nb = ep // _B

    x128 = _sc_gather(nodes2, sp >> 1, ep)

    kflat = jnp.transpose(kernels, (1, 0, 2)).reshape(in_f, rf)
    kbig = jnp.concatenate([kflat, kflat], axis=0).astype(jnp.bfloat16)
    kgrp = _regroup(kbig, _G, num_rel, out_f)

    out = _tc_messages(x128, sp.reshape(nb, 1, _B), tp.reshape(nb, 1, _B),
                       kgrp, e, ep, in_f, rf, out_f)
    return out.T
